# Initial kernel scaffold; baseline (speedup 1.0000x reference)
import jax
import jax.numpy as jnp
from jax.experimental import pallas as pl


def kernel(tokens, token_table, pos_table):
    raise NotImplementedError("write your pallas kernel here")


# speedup vs baseline: 3.7284x; 3.7284x over previous
import functools
import math

import jax
import jax.numpy as jnp
from jax import lax
from jax.experimental import pallas as pl
from jax.experimental.pallas import tpu as pltpu
from jax.experimental.pallas import tpu_sc as plsc

_LANES = 16
_NBUF = 4


def _build(batch, seq, dim, vocab):
    nc, ns = 2, 16
    nw = nc * ns
    spw = seq // nw
    scale = math.sqrt(dim)
    mesh = plsc.VectorSubcoreMesh(
        core_axis_name="c", subcore_axis_name="s", num_cores=nc, num_subcores=ns
    )

    scratch = [
        pltpu.VMEM((batch, spw), jnp.int32),
        pltpu.VMEM((spw, dim), jnp.float32),
    ]
    scratch += [pltpu.VMEM((spw, dim), jnp.float32) for _ in range(_NBUF)]
    scratch += [pltpu.SemaphoreType.DMA for _ in range(2 * _NBUF)]

    @functools.partial(
        pl.kernel,
        out_type=jax.ShapeDtypeStruct((batch, seq, dim), jnp.float32),
        mesh=mesh,
        scratch_types=scratch,
    )
    def emb(idx_hbm, pos_hbm, table_hbm, out_hbm, idx_v, pos_v, *bufs_sems):
        bufs = bufs_sems[:_NBUF]
        gsem = bufs_sems[_NBUF:2 * _NBUF]
        ssem = bufs_sems[2 * _NBUF:]

        w = lax.axis_index("s") * nc + lax.axis_index("c")
        s0 = w * spw

        pltpu.sync_copy(idx_hbm.at[w], idx_v)
        pltpu.sync_copy(pos_hbm.at[pl.ds(s0, spw)], pos_v)

        def gather(b, t):
            pltpu.async_copy(table_hbm.at[idx_v.at[b]], bufs[t], gsem[t])

        def wait_gather(t):
            pltpu.make_async_copy(table_hbm.at[idx_v.at[0]], bufs[t],
                                  gsem[t]).wait()

        def store(b, t):
            pltpu.async_copy(bufs[t], out_hbm.at[b, pl.ds(s0, spw), :],
                             ssem[t])

        def wait_store(t):
            pltpu.make_async_copy(bufs[t], out_hbm.at[0, pl.ds(s0, spw), :],
                                  ssem[t]).wait()

        def compute(t):
            buf = bufs[t]

            def row(i, _):
                for j in range(dim // _LANES):
                    sl = pl.ds(j * _LANES, _LANES)
                    buf[i, sl] = buf[i, sl] * scale + pos_v[i, sl]
                return ()

            lax.fori_loop(0, spw, row, ())

        for t in range(_NBUF):
            gather(t, t)

        @pl.loop(0, batch // _NBUF)
        def _block(g):
            for t in range(_NBUF):
                b = g * _NBUF + t
                wait_gather(t)
                compute(t)
                store(b, t)
                nb = b + _NBUF - 1
                tn = (t + _NBUF - 1) % _NBUF

                @pl.when(jnp.logical_and(nb >= _NBUF, nb < batch))
                def _():
                    wait_store(tn)
                    gather(nb, tn)

        for t in range(_NBUF):
            wait_store(t)

    return emb


def kernel(tokens, token_table, pos_table):
    batch, seq = tokens.shape
    vocab, dim = token_table.shape
    nw = 32
    spw = seq // nw
    idx = tokens.astype(jnp.int32).reshape(batch, nw, spw).transpose(1, 0, 2)
    emb = _build(batch, seq, dim, vocab)
    return emb(idx, pos_table[:seq], token_table)
